# Initial kernel scaffold; baseline (speedup 1.0000x reference)
import jax
import jax.numpy as jnp
from jax.experimental import pallas as pl


def kernel(feat, label):
    raise NotImplementedError("write your pallas kernel here")


# speedup vs baseline: 2.0915x; 2.0915x over previous
import functools

import jax
import jax.numpy as jnp
from jax import lax
from jax.experimental import pallas as pl

K_WAY = 5
N_SHOT = 5
D = 4096
NQ = 16384
N_AUG = 70
TOPK = 50
NS = K_WAY + N_AUG

QT = 1024
NT = NQ // QT


def _logsumexp_rows(z):
    m = jnp.max(z, axis=1, keepdims=True)
    return m + jnp.log(jnp.sum(jnp.exp(z - m), axis=1, keepdims=True))


def _pass1_body(sup_ref, q_ref, tgt_ref, stats_ref, cnt_ref, loss_ref):
    i = pl.program_id(0)

    @pl.when(i == 0)
    def _init():
        stats_ref[...] = jnp.zeros_like(stats_ref)
        cnt_ref[...] = jnp.zeros_like(cnt_ref)
        loss_ref[...] = jnp.zeros_like(loss_ref)

    sup = sup_ref[...]
    proto = jnp.mean(sup.reshape(K_WAY, N_SHOT, D), axis=1)
    pn = jnp.sum(proto * proto, axis=1)

    q = q_ref[...]
    qsq = q * q
    qn = jnp.sum(qsq, axis=1, keepdims=True)
    qp = lax.dot_general(q, proto, (((1,), (1,)), ((), ())),
                         preferred_element_type=jnp.float32)
    d2 = qn + pn[None, :] - 2.0 * qp
    d_e = jnp.sqrt(jnp.clip(d2, 1e-12))

    z = -d_e
    zmax = jnp.max(z, axis=1, keepdims=True)
    iota_k = lax.broadcasted_iota(jnp.int32, (QT, K_WAY), 1)
    cand = jnp.where(z >= zmax, iota_k, K_WAY)
    pred = jnp.min(cand, axis=1, keepdims=True)
    onehot = (iota_k == pred).astype(jnp.float32)

    cnt_ref[...] += jnp.sum(onehot, axis=0, keepdims=True)
    qsum = lax.dot_general(onehot, q, (((0,), (0,)), ((), ())),
                           preferred_element_type=jnp.float32)
    qsqs = lax.dot_general(onehot, qsq, (((0,), (0,)), ((), ())),
                           preferred_element_type=jnp.float32)
    stats_ref[0] += qsum
    stats_ref[1] += qsqs

    a = z - _logsumexp_rows(z)
    b = a - _logsumexp_rows(a)
    loss_ref[...] += -jnp.sum(tgt_ref[...] * b, keepdims=True).reshape(1, 1)


def _mid_body(sup_ref, stats_ref, cnt_ref, noise_ref, pnew_ref):
    sup = sup_ref[...].reshape(K_WAY, N_SHOT, D)
    proto = jnp.mean(sup, axis=1)
    cnt = cnt_ref[...].reshape(K_WAY, 1) + float(N_SHOT)
    s_sum = jnp.sum(sup, axis=1) + stats_ref[0]
    s_sq = jnp.sum(sup * sup, axis=1) + stats_ref[1]
    mean_c = s_sum / cnt
    var_c = (s_sq - cnt * mean_c * mean_c) / (cnt - 1.0)
    std_c = jnp.sqrt(jnp.clip(var_c, 1e-12))

    samples = mean_c[:, None, :] + std_c[:, None, :] * noise_ref[...]
    s_new = jnp.concatenate([sup, samples], axis=1)

    diff = s_new - proto[:, None, :]
    d2 = jnp.sum(diff * diff, axis=-1)
    d = jnp.sqrt(jnp.clip(d2, 1e-12))

    di = d[:, :, None]
    dj = d[:, None, :]
    ii = lax.broadcasted_iota(jnp.int32, (K_WAY, NS, NS), 1)
    jj = lax.broadcasted_iota(jnp.int32, (K_WAY, NS, NS), 2)
    cmp = (dj < di) | ((dj == di) & (jj < ii))
    rank = jnp.sum(cmp.astype(jnp.float32), axis=2)
    sel = (rank < float(TOPK)).astype(jnp.float32)

    pnew = lax.dot_general(sel, s_new, (((1,), (1,)), ((0,), (0,))),
                           preferred_element_type=jnp.float32)
    pnew_ref[...] = pnew * (1.0 / TOPK)


def _pass2_body(pnew_ref, q_ref, out_ref):
    pnew = pnew_ref[...]
    pn = jnp.sum(pnew * pnew, axis=1)
    q = q_ref[...]
    qn = jnp.sum(q * q, axis=1, keepdims=True)
    qp = lax.dot_general(q, pnew, (((1,), (1,)), ((), ())),
                         preferred_element_type=jnp.float32)
    d2 = qn + pn[None, :] - 2.0 * qp
    d_new = jnp.sqrt(jnp.clip(d2, 1e-12))
    z = -d_new
    m = jnp.max(z, axis=1, keepdims=True)
    e = jnp.exp(z - m)
    out_ref[...] = e / jnp.sum(e, axis=1, keepdims=True)


@jax.jit
def kernel(feat, label):
    support = feat[: K_WAY * N_SHOT]
    queries = feat[K_WAY * N_SHOT:]
    tgt_oh = jax.nn.one_hot(label[1], K_WAY, dtype=jnp.float32)

    stats, cnt, loss_sum = pl.pallas_call(
        _pass1_body,
        grid=(NT,),
        in_specs=[
            pl.BlockSpec((K_WAY * N_SHOT, D), lambda i: (0, 0)),
            pl.BlockSpec((QT, D), lambda i: (i, 0)),
            pl.BlockSpec((QT, K_WAY), lambda i: (i, 0)),
        ],
        out_specs=[
            pl.BlockSpec((2, K_WAY, D), lambda i: (0, 0, 0)),
            pl.BlockSpec((1, K_WAY), lambda i: (0, 0)),
            pl.BlockSpec((1, 1), lambda i: (0, 0)),
        ],
        out_shape=[
            jax.ShapeDtypeStruct((2, K_WAY, D), jnp.float32),
            jax.ShapeDtypeStruct((1, K_WAY), jnp.float32),
            jax.ShapeDtypeStruct((1, 1), jnp.float32),
        ],
    )(support, queries, tgt_oh)

    noise = jax.random.normal(jax.random.key(42), (K_WAY, N_AUG, D),
                              dtype=jnp.float32)

    proto_new = pl.pallas_call(
        _mid_body,
        out_shape=jax.ShapeDtypeStruct((K_WAY, D), jnp.float32),
    )(support, stats, cnt, noise)

    y_pred = pl.pallas_call(
        _pass2_body,
        grid=(NT,),
        in_specs=[
            pl.BlockSpec((K_WAY, D), lambda i: (0, 0)),
            pl.BlockSpec((QT, D), lambda i: (i, 0)),
        ],
        out_specs=pl.BlockSpec((QT, K_WAY), lambda i: (i, 0)),
        out_shape=jax.ShapeDtypeStruct((NQ, K_WAY), jnp.float32),
    )(proto_new, queries)

    loss = loss_sum[0, 0] / NQ
    return (y_pred, loss)


# speedup vs baseline: 3.0752x; 1.4703x over previous
import jax
import jax.numpy as jnp
from jax import lax
from jax.experimental import pallas as pl
from jax.experimental.pallas import tpu as pltpu

K_WAY = 5
N_SHOT = 5
NSUP = K_WAY * N_SHOT
D = 4096
NQ = 16384
N_AUG = 70
TOPK = 50
NS = K_WAY + N_AUG

QT = 1024
NT1 = NQ // QT
TAIL = 32
NT2 = (NSUP + NQ + QT - 1) // QT


def _logsumexp_rows(z):
    m = jnp.max(z, axis=1, keepdims=True)
    return m + jnp.log(jnp.sum(jnp.exp(z - m), axis=1, keepdims=True))


def _accumulate(q, tgt, proto, pn, valid, stats_ref, cnt_ref, loss_ref):
    rows = q.shape[0]
    qsq = q * q
    qn = jnp.sum(qsq, axis=1, keepdims=True)
    qp = lax.dot_general(q, proto, (((1,), (1,)), ((), ())),
                         preferred_element_type=jnp.float32)
    d2 = qn + pn[None, :] - 2.0 * qp
    d_e = jnp.sqrt(jnp.clip(d2, 1e-12))

    z = -d_e
    zmax = jnp.max(z, axis=1, keepdims=True)
    iota_k = lax.broadcasted_iota(jnp.int32, (rows, K_WAY), 1)
    cand = jnp.where(z >= zmax, iota_k, K_WAY)
    pred = jnp.min(cand, axis=1, keepdims=True)
    onehot = jnp.where((iota_k == pred) & valid, 1.0, 0.0)

    cnt_ref[...] += jnp.sum(onehot, axis=0, keepdims=True)
    stats_ref[0] += lax.dot_general(onehot, q, (((0,), (0,)), ((), ())),
                                    preferred_element_type=jnp.float32)
    stats_ref[1] += lax.dot_general(onehot, qsq, (((0,), (0,)), ((), ())),
                                    preferred_element_type=jnp.float32)

    a = z - _logsumexp_rows(z)
    b = a - _logsumexp_rows(a)
    loss_ref[...] += -jnp.sum(tgt * b, keepdims=True).reshape(1, 1)


def _pass1_body(q_ref, tail_ref, tgt_ref, tgt_tail_ref,
                stats_ref, cnt_ref, loss_ref, proto_ref):
    i = pl.program_id(0)

    @pl.when(i == 0)
    def _init():
        stats_ref[...] = jnp.zeros_like(stats_ref)
        cnt_ref[...] = jnp.zeros_like(cnt_ref)
        loss_ref[...] = jnp.zeros_like(loss_ref)
        sup = q_ref[:NSUP, :].reshape(K_WAY, N_SHOT, D)
        proto_ref[:K_WAY] = jnp.mean(sup, axis=1)

    proto = proto_ref[:K_WAY]
    pn = jnp.sum(proto * proto, axis=1)

    q = q_ref[...]
    r0 = lax.broadcasted_iota(jnp.int32, (QT, 1), 0) + i * QT
    valid = r0 >= NSUP
    _accumulate(q, tgt_ref[...], proto, pn, valid, stats_ref, cnt_ref,
                loss_ref)

    @pl.when(i == NT1 - 1)
    def _tail():
        rloc = lax.broadcasted_iota(jnp.int32, (TAIL, 1), 0)
        tvalid = rloc < NSUP
        qt = jnp.where(tvalid, tail_ref[...], 0.0)
        _accumulate(qt, tgt_tail_ref[...], proto, pn, tvalid,
                    stats_ref, cnt_ref, loss_ref)


def _mid_body(sup_ref, stats_ref, cnt_ref, noise_ref, pnew_ref):
    sup = sup_ref[:NSUP, :].reshape(K_WAY, N_SHOT, D)
    proto = jnp.mean(sup, axis=1)
    cnt = cnt_ref[...].reshape(K_WAY, 1) + float(N_SHOT)
    s_sum = jnp.sum(sup, axis=1) + stats_ref[0]
    s_sq = jnp.sum(sup * sup, axis=1) + stats_ref[1]
    mean_c = s_sum / cnt
    var_c = (s_sq - cnt * mean_c * mean_c) / (cnt - 1.0)
    std_c = jnp.sqrt(jnp.clip(var_c, 1e-12))

    samples = mean_c[:, None, :] + std_c[:, None, :] * noise_ref[...]
    s_new = jnp.concatenate([sup, samples], axis=1)

    diff = s_new - proto[:, None, :]
    d2 = jnp.sum(diff * diff, axis=-1)
    d = jnp.sqrt(jnp.clip(d2, 1e-12))

    di = d[:, :, None]
    dj = d[:, None, :]
    ii = lax.broadcasted_iota(jnp.int32, (K_WAY, NS, NS), 1)
    jj = lax.broadcasted_iota(jnp.int32, (K_WAY, NS, NS), 2)
    cmp = (dj < di) | ((dj == di) & (jj < ii))
    rank = jnp.sum(cmp.astype(jnp.float32), axis=2)
    sel = (rank < float(TOPK)).astype(jnp.float32)

    pnew = lax.dot_general(sel, s_new, (((1,), (1,)), ((0,), (0,))),
                           preferred_element_type=jnp.float32)
    pnew_ref[...] = pnew * (1.0 / TOPK)


def _pass2_body(pnew_ref, q_ref, out_ref):
    pnew = pnew_ref[...]
    pn = jnp.sum(pnew * pnew, axis=1)
    q = q_ref[...]
    qn = jnp.sum(q * q, axis=1, keepdims=True)
    qp = lax.dot_general(q, pnew, (((1,), (1,)), ((), ())),
                         preferred_element_type=jnp.float32)
    d2 = qn + pn[None, :] - 2.0 * qp
    z = -jnp.sqrt(jnp.clip(d2, 1e-12))
    m = jnp.max(z, axis=1, keepdims=True)
    e = jnp.exp(z - m)
    out_ref[...] = e / jnp.sum(e, axis=1, keepdims=True)


@jax.jit
def kernel(feat, label):
    tgt_oh = jax.nn.one_hot(label[1], K_WAY, dtype=jnp.float32)
    tgt_pad = jnp.pad(tgt_oh, ((NSUP, TAIL - NSUP), (0, 0)))

    stats, cnt, loss_sum = pl.pallas_call(
        _pass1_body,
        grid=(NT1,),
        in_specs=[
            pl.BlockSpec((QT, D), lambda i: (i, 0)),
            pl.BlockSpec((TAIL, D), lambda i: (NQ // TAIL, 0)),
            pl.BlockSpec((QT, K_WAY), lambda i: (i, 0)),
            pl.BlockSpec((TAIL, K_WAY), lambda i: (NQ // TAIL, 0)),
        ],
        out_specs=[
            pl.BlockSpec((2, K_WAY, D), lambda i: (0, 0, 0)),
            pl.BlockSpec((1, K_WAY), lambda i: (0, 0)),
            pl.BlockSpec((1, 1), lambda i: (0, 0)),
        ],
        out_shape=[
            jax.ShapeDtypeStruct((2, K_WAY, D), jnp.float32),
            jax.ShapeDtypeStruct((1, K_WAY), jnp.float32),
            jax.ShapeDtypeStruct((1, 1), jnp.float32),
        ],
        scratch_shapes=[pltpu.VMEM((8, D), jnp.float32)],
    )(feat, feat, tgt_pad, tgt_pad)

    noise = jax.random.normal(jax.random.key(42), (K_WAY, N_AUG, D),
                              dtype=jnp.float32)

    proto_new = pl.pallas_call(
        _mid_body,
        in_specs=[
            pl.BlockSpec((TAIL, D), lambda i: (0, 0)),
            pl.BlockSpec((2, K_WAY, D), lambda i: (0, 0, 0)),
            pl.BlockSpec((1, K_WAY), lambda i: (0, 0)),
            pl.BlockSpec((K_WAY, N_AUG, D), lambda i: (0, 0, 0)),
        ],
        out_specs=pl.BlockSpec((K_WAY, D), lambda i: (0, 0)),
        out_shape=jax.ShapeDtypeStruct((K_WAY, D), jnp.float32),
        grid=(1,),
    )(feat, stats, cnt, noise)

    y_pad = pl.pallas_call(
        _pass2_body,
        grid=(NT2,),
        in_specs=[
            pl.BlockSpec((K_WAY, D), lambda i: (0, 0)),
            pl.BlockSpec((QT, D), lambda i: (i, 0)),
        ],
        out_specs=pl.BlockSpec((QT, K_WAY), lambda i: (i, 0)),
        out_shape=jax.ShapeDtypeStruct((NT2 * QT, K_WAY), jnp.float32),
    )(proto_new, feat)

    y_pred = lax.slice(y_pad, (NSUP, 0), (NSUP + NQ, K_WAY))
    loss = loss_sum[0, 0] / NQ
    return (y_pred, loss)


# speedup vs baseline: 3.5102x; 1.1415x over previous
import jax
import jax.numpy as jnp
from jax import lax
from jax.experimental import pallas as pl
from jax.experimental.pallas import tpu as pltpu

K_WAY = 5
N_SHOT = 5
NSUP = K_WAY * N_SHOT
D = 4096
NQ = 16384
N_AUG = 70
TOPK = 50
NS = K_WAY + N_AUG

QT = 1024
NT1 = NQ // QT
TAIL = 32
NT2 = (NSUP + NQ + QT - 1) // QT

_NOISE = jax.random.normal(jax.random.key(42), (K_WAY, N_AUG, D),
                           dtype=jnp.float32)


def _logsumexp_rows(z):
    m = jnp.max(z, axis=1, keepdims=True)
    return m + jnp.log(jnp.sum(jnp.exp(z - m), axis=1, keepdims=True))


def _accumulate(q, tgt, proto, pn, valid, stats_ref, cnt_ref, loss_ref):
    rows = q.shape[0]
    qsq = q * q
    qn = jnp.sum(qsq, axis=1, keepdims=True)
    qp = lax.dot_general(q, proto, (((1,), (1,)), ((), ())),
                         preferred_element_type=jnp.float32)
    d2 = qn + pn[None, :] - 2.0 * qp
    d_e = jnp.sqrt(jnp.clip(d2, 1e-12))

    z = -d_e
    zmax = jnp.max(z, axis=1, keepdims=True)
    iota_k = lax.broadcasted_iota(jnp.int32, (rows, K_WAY), 1)
    cand = jnp.where(z >= zmax, iota_k, K_WAY)
    pred = jnp.min(cand, axis=1, keepdims=True)
    onehot = jnp.where((iota_k == pred) & valid, 1.0, 0.0)

    cnt_ref[...] += jnp.sum(onehot, axis=0, keepdims=True)
    stats_ref[0] += lax.dot_general(onehot, q, (((0,), (0,)), ((), ())),
                                    preferred_element_type=jnp.float32)
    stats_ref[1] += lax.dot_general(onehot, qsq, (((0,), (0,)), ((), ())),
                                    preferred_element_type=jnp.float32)

    a = z - _logsumexp_rows(z)
    b = a - _logsumexp_rows(a)
    loss_ref[...] += -jnp.sum(tgt * b, keepdims=True).reshape(1, 1)


def _pass1_body(q_ref, tail_ref, tgt_ref, tgt_tail_ref,
                stats_ref, cnt_ref, loss_ref, proto_ref):
    i = pl.program_id(0)

    @pl.when(i == 0)
    def _init():
        stats_ref[...] = jnp.zeros_like(stats_ref)
        cnt_ref[...] = jnp.zeros_like(cnt_ref)
        loss_ref[...] = jnp.zeros_like(loss_ref)
        sup = q_ref[:NSUP, :].reshape(K_WAY, N_SHOT, D)
        proto_ref[:K_WAY] = jnp.mean(sup, axis=1)

    proto = proto_ref[:K_WAY]
    pn = jnp.sum(proto * proto, axis=1)

    q = q_ref[...]
    r0 = lax.broadcasted_iota(jnp.int32, (QT, 1), 0) + i * QT
    valid = r0 >= NSUP
    _accumulate(q, tgt_ref[...], proto, pn, valid, stats_ref, cnt_ref,
                loss_ref)

    @pl.when(i == NT1 - 1)
    def _tail():
        rloc = lax.broadcasted_iota(jnp.int32, (TAIL, 1), 0)
        tvalid = rloc < NSUP
        qt = jnp.where(tvalid, tail_ref[...], 0.0)
        _accumulate(qt, tgt_tail_ref[...], proto, pn, tvalid,
                    stats_ref, cnt_ref, loss_ref)


def _refresh_protos(sup_ref, stats_ref, cnt_ref, noise_ref, pnew_ref):
    sup = sup_ref[:NSUP, :].reshape(K_WAY, N_SHOT, D)
    proto = jnp.mean(sup, axis=1)
    cnt = cnt_ref[...].reshape(K_WAY, 1) + float(N_SHOT)
    s_sum = jnp.sum(sup, axis=1) + stats_ref[0]
    s_sq = jnp.sum(sup * sup, axis=1) + stats_ref[1]
    mean_c = s_sum / cnt
    var_c = (s_sq - cnt * mean_c * mean_c) / (cnt - 1.0)
    std_c = jnp.sqrt(jnp.clip(var_c, 1e-12))

    noise = noise_ref[...]
    delta = mean_c - proto
    u = delta * std_c
    v = std_c * std_c
    dn2 = jnp.sum(delta * delta, axis=1, keepdims=True)
    bdims = (((2,), (1,)), ((0,), (0,)))
    cross = lax.dot_general(noise, u, bdims,
                            preferred_element_type=jnp.float32)
    quad = lax.dot_general(noise * noise, v, bdims,
                           preferred_element_type=jnp.float32)
    d2_smp = dn2 + 2.0 * cross + quad

    dsup = sup - proto[:, None, :]
    d2_sup = jnp.sum(dsup * dsup, axis=2)
    d = jnp.sqrt(jnp.clip(jnp.concatenate([d2_sup, d2_smp], axis=1),
                          1e-12))

    di = d[:, :, None]
    dj = d[:, None, :]
    ii = lax.broadcasted_iota(jnp.int32, (K_WAY, NS, NS), 1)
    jj = lax.broadcasted_iota(jnp.int32, (K_WAY, NS, NS), 2)
    cmp = (dj < di) | ((dj == di) & (jj < ii))
    rank = jnp.sum(cmp.astype(jnp.float32), axis=2)
    sel = (rank < float(TOPK)).astype(jnp.float32)
    sel_sup = sel[:, :N_SHOT]
    sel_smp = sel[:, N_SHOT:]
    nsel = jnp.sum(sel_smp, axis=1, keepdims=True)

    sup_part = lax.dot_general(sel_sup, sup, (((1,), (1,)), ((0,), (0,))),
                               preferred_element_type=jnp.float32)
    noise_part = lax.dot_general(sel_smp, noise, (((1,), (1,)), ((0,), (0,))),
                                 preferred_element_type=jnp.float32)
    pnew = sup_part + nsel * mean_c + std_c * noise_part
    pnew_ref[:K_WAY] = pnew * (1.0 / TOPK)


def _pass2_body(sup_ref, stats_ref, cnt_ref, noise_ref, q_ref,
                out_ref, pnew_ref):
    i = pl.program_id(0)

    @pl.when(i == 0)
    def _mid():
        _refresh_protos(sup_ref, stats_ref, cnt_ref, noise_ref, pnew_ref)

    pnew = pnew_ref[:K_WAY]
    pn = jnp.sum(pnew * pnew, axis=1)
    q = q_ref[...]
    qn = jnp.sum(q * q, axis=1, keepdims=True)
    qp = lax.dot_general(q, pnew, (((1,), (1,)), ((), ())),
                         preferred_element_type=jnp.float32)
    d2 = qn + pn[None, :] - 2.0 * qp
    z = -jnp.sqrt(jnp.clip(d2, 1e-12))
    m = jnp.max(z, axis=1, keepdims=True)
    e = jnp.exp(z - m)
    out_ref[...] = e / jnp.sum(e, axis=1, keepdims=True)


@jax.jit
def kernel(feat, label):
    tgt_oh = jax.nn.one_hot(label[1], K_WAY, dtype=jnp.float32)
    tgt_pad = jnp.pad(tgt_oh, ((NSUP, TAIL - NSUP), (0, 0)))

    stats, cnt, loss_sum = pl.pallas_call(
        _pass1_body,
        grid=(NT1,),
        in_specs=[
            pl.BlockSpec((QT, D), lambda i: (i, 0)),
            pl.BlockSpec((TAIL, D), lambda i: (NQ // TAIL, 0)),
            pl.BlockSpec((QT, K_WAY), lambda i: (i, 0)),
            pl.BlockSpec((TAIL, K_WAY), lambda i: (NQ // TAIL, 0)),
        ],
        out_specs=[
            pl.BlockSpec((2, K_WAY, D), lambda i: (0, 0, 0)),
            pl.BlockSpec((1, K_WAY), lambda i: (0, 0)),
            pl.BlockSpec((1, 1), lambda i: (0, 0)),
        ],
        out_shape=[
            jax.ShapeDtypeStruct((2, K_WAY, D), jnp.float32),
            jax.ShapeDtypeStruct((1, K_WAY), jnp.float32),
            jax.ShapeDtypeStruct((1, 1), jnp.float32),
        ],
        scratch_shapes=[pltpu.VMEM((8, D), jnp.float32)],
    )(feat, feat, tgt_pad, tgt_pad)

    y_pad = pl.pallas_call(
        _pass2_body,
        grid=(NT2,),
        in_specs=[
            pl.BlockSpec((TAIL, D), lambda i: (0, 0)),
            pl.BlockSpec((2, K_WAY, D), lambda i: (0, 0, 0)),
            pl.BlockSpec((1, K_WAY), lambda i: (0, 0)),
            pl.BlockSpec((K_WAY, N_AUG, D), lambda i: (0, 0, 0)),
            pl.BlockSpec((QT, D), lambda i: (i, 0)),
        ],
        out_specs=pl.BlockSpec((QT, K_WAY), lambda i: (i, 0)),
        out_shape=jax.ShapeDtypeStruct((NT2 * QT, K_WAY), jnp.float32),
        scratch_shapes=[pltpu.VMEM((8, D), jnp.float32)],
    )(feat, stats, cnt, _NOISE, feat)

    y_pred = lax.slice(y_pad, (NSUP, 0), (NSUP + NQ, K_WAY))
    loss = loss_sum[0, 0] / NQ
    return (y_pred, loss)


# speedup vs baseline: 3.5475x; 1.0106x over previous
import jax
import jax.numpy as jnp
from jax import lax
from jax.experimental import pallas as pl
from jax.experimental.pallas import tpu as pltpu

K_WAY = 5
N_SHOT = 5
NSUP = K_WAY * N_SHOT
D = 4096
NQ = 16384
N_AUG = 70
TOPK = 50
NS = K_WAY + N_AUG

QT = 512
NT1 = NQ // QT
TAIL = 32

_NOISE = jax.random.normal(jax.random.key(42), (K_WAY, N_AUG, D),
                           dtype=jnp.float32)


def _logsumexp_rows(z):
    m = jnp.max(z, axis=1, keepdims=True)
    return m + jnp.log(jnp.sum(jnp.exp(z - m), axis=1, keepdims=True))


def _accumulate(q, tgt, proto, pn, valid, stats_scr, cnt_scr, loss_ref):
    rows = q.shape[0]
    qsq = q * q
    qn = jnp.sum(qsq, axis=1, keepdims=True)
    qp = lax.dot_general(q, proto, (((1,), (1,)), ((), ())),
                         preferred_element_type=jnp.float32)
    d2 = qn + pn[None, :] - 2.0 * qp
    d_e = jnp.sqrt(jnp.clip(d2, 1e-12))

    z = -d_e
    zmax = jnp.max(z, axis=1, keepdims=True)
    iota_k = lax.broadcasted_iota(jnp.int32, (rows, K_WAY), 1)
    cand = jnp.where(z >= zmax, iota_k, K_WAY)
    pred = jnp.min(cand, axis=1, keepdims=True)
    onehot = jnp.where((iota_k == pred) & valid, 1.0, 0.0)

    cnt_scr[...] += jnp.sum(onehot, axis=0, keepdims=True)
    stats_scr[0] += lax.dot_general(onehot, q, (((0,), (0,)), ((), ())),
                                    preferred_element_type=jnp.float32)
    stats_scr[1] += lax.dot_general(onehot, qsq, (((0,), (0,)), ((), ())),
                                    preferred_element_type=jnp.float32)

    a = z - _logsumexp_rows(z)
    b = a - _logsumexp_rows(a)
    loss_ref[...] += -jnp.sum(tgt * b, keepdims=True).reshape(1, 1)


def _refresh_protos(sup_ref, stats_scr, cnt_scr, noise_ref, pnew_scr):
    sup = sup_ref[:NSUP, :].reshape(K_WAY, N_SHOT, D)
    proto = jnp.mean(sup, axis=1)
    cnt = cnt_scr[...].reshape(K_WAY, 1) + float(N_SHOT)
    s_sum = jnp.sum(sup, axis=1) + stats_scr[0]
    s_sq = jnp.sum(sup * sup, axis=1) + stats_scr[1]
    mean_c = s_sum / cnt
    var_c = (s_sq - cnt * mean_c * mean_c) / (cnt - 1.0)
    std_c = jnp.sqrt(jnp.clip(var_c, 1e-12))

    noise = noise_ref[...]
    delta = mean_c - proto
    u = delta * std_c
    v = std_c * std_c
    dn2 = jnp.sum(delta * delta, axis=1, keepdims=True)
    bdims = (((2,), (1,)), ((0,), (0,)))
    cross = lax.dot_general(noise, u, bdims,
                            preferred_element_type=jnp.float32)
    quad = lax.dot_general(noise * noise, v, bdims,
                           preferred_element_type=jnp.float32)
    d2_smp = dn2 + 2.0 * cross + quad

    dsup = sup - proto[:, None, :]
    d2_sup = jnp.sum(dsup * dsup, axis=2)
    d = jnp.sqrt(jnp.clip(jnp.concatenate([d2_sup, d2_smp], axis=1),
                          1e-12))

    di = d[:, :, None]
    dj = d[:, None, :]
    ii = lax.broadcasted_iota(jnp.int32, (K_WAY, NS, NS), 1)
    jj = lax.broadcasted_iota(jnp.int32, (K_WAY, NS, NS), 2)
    cmp = (dj < di) | ((dj == di) & (jj < ii))
    rank = jnp.sum(cmp.astype(jnp.float32), axis=2)
    sel = (rank < float(TOPK)).astype(jnp.float32)
    sel_sup = sel[:, :N_SHOT]
    sel_smp = sel[:, N_SHOT:]
    nsel = jnp.sum(sel_smp, axis=1, keepdims=True)

    sup_part = lax.dot_general(sel_sup, sup, (((1,), (1,)), ((0,), (0,))),
                               preferred_element_type=jnp.float32)
    noise_part = lax.dot_general(sel_smp, noise, (((1,), (1,)), ((0,), (0,))),
                                 preferred_element_type=jnp.float32)
    pnew = sup_part + nsel * mean_c + std_c * noise_part
    pnew_scr[:K_WAY] = pnew * (1.0 / TOPK)


def _softmax_rows(q, pnew, pn):
    qn = jnp.sum(q * q, axis=1, keepdims=True)
    qp = lax.dot_general(q, pnew, (((1,), (1,)), ((), ())),
                         preferred_element_type=jnp.float32)
    d2 = qn + pn[None, :] - 2.0 * qp
    z = -jnp.sqrt(jnp.clip(d2, 1e-12))
    m = jnp.max(z, axis=1, keepdims=True)
    e = jnp.exp(z - m)
    return e / jnp.sum(e, axis=1, keepdims=True)


def _body(q_ref, tailf_ref, tgt_ref, tgtt_ref, noise_ref,
          loss_ref, y_ref, yt_ref,
          proto_scr, stats_scr, cnt_scr, pnew_scr):
    i = pl.program_id(0)

    @pl.when(i == 0)
    def _init():
        stats_scr[...] = jnp.zeros_like(stats_scr)
        cnt_scr[...] = jnp.zeros_like(cnt_scr)
        loss_ref[...] = jnp.zeros_like(loss_ref)
        sup = q_ref[:NSUP, :].reshape(K_WAY, N_SHOT, D)
        proto_scr[:K_WAY] = jnp.mean(sup, axis=1)

    @pl.when(i < NT1)
    def _phase1():
        proto = proto_scr[:K_WAY]
        pn = jnp.sum(proto * proto, axis=1)
        q = q_ref[...]
        r0 = lax.broadcasted_iota(jnp.int32, (QT, 1), 0) + i * QT
        valid = r0 >= NSUP
        _accumulate(q, tgt_ref[...], proto, pn, valid,
                    stats_scr, cnt_scr, loss_ref)

        @pl.when(i == NT1 - 1)
        def _tail():
            rloc = lax.broadcasted_iota(jnp.int32, (TAIL, 1), 0)
            tvalid = rloc < NSUP
            qt = jnp.where(tvalid, tailf_ref[...], 0.0)
            _accumulate(qt, tgtt_ref[...], proto, pn, tvalid,
                        stats_scr, cnt_scr, loss_ref)

    @pl.when(i == NT1)
    def _mid():
        _refresh_protos(q_ref, stats_scr, cnt_scr, noise_ref, pnew_scr)

    @pl.when(i >= NT1)
    def _phase2():
        pnew = pnew_scr[:K_WAY]
        pn = jnp.sum(pnew * pnew, axis=1)
        y_ref[...] = _softmax_rows(q_ref[...], pnew, pn)

        @pl.when(i == 2 * NT1 - 1)
        def _tail2():
            yt_ref[...] = _softmax_rows(tailf_ref[...], pnew, pn)


@jax.jit
def kernel(feat, label):
    tgt_oh = jax.nn.one_hot(label[1], K_WAY, dtype=jnp.float32)
    tgt_pad = jnp.pad(tgt_oh, ((NSUP, TAIL - NSUP), (0, 0)))

    loss_sum, y_main, y_tail = pl.pallas_call(
        _body,
        grid=(2 * NT1,),
        in_specs=[
            pl.BlockSpec((QT, D),
                         lambda i: (jnp.where(i < NT1, i, i - NT1), 0)),
            pl.BlockSpec((TAIL, D), lambda i: (NQ // TAIL, 0)),
            pl.BlockSpec((QT, K_WAY), lambda i: (jnp.minimum(i, NT1 - 1), 0)),
            pl.BlockSpec((TAIL, K_WAY), lambda i: (NQ // TAIL, 0)),
            pl.BlockSpec((K_WAY, N_AUG, D), lambda i: (0, 0, 0)),
        ],
        out_specs=[
            pl.BlockSpec((1, 1), lambda i: (0, 0)),
            pl.BlockSpec((QT, K_WAY),
                         lambda i: (jnp.where(i < NT1, 0, i - NT1), 0)),
            pl.BlockSpec((TAIL, K_WAY), lambda i: (0, 0)),
        ],
        out_shape=[
            jax.ShapeDtypeStruct((1, 1), jnp.float32),
            jax.ShapeDtypeStruct((NQ, K_WAY), jnp.float32),
            jax.ShapeDtypeStruct((TAIL, K_WAY), jnp.float32),
        ],
        scratch_shapes=[
            pltpu.VMEM((8, D), jnp.float32),
            pltpu.VMEM((2, K_WAY, D), jnp.float32),
            pltpu.VMEM((1, K_WAY), jnp.float32),
            pltpu.VMEM((8, D), jnp.float32),
        ],
    )(feat, feat, tgt_pad, tgt_pad, _NOISE)

    y_pred = jnp.concatenate(
        [lax.slice(y_main, (NSUP, 0), (NQ, K_WAY)),
         lax.slice(y_tail, (0, 0), (NSUP, K_WAY))], axis=0)
    loss = loss_sum[0, 0] / NQ
    return (y_pred, loss)


# speedup vs baseline: 3.6281x; 1.0227x over previous
import jax
import jax.numpy as jnp
from jax import lax
from jax.experimental import pallas as pl
from jax.experimental.pallas import tpu as pltpu

K_WAY = 5
N_SHOT = 5
NSUP = K_WAY * N_SHOT
D = 4096
NQ = 16384
N_AUG = 70
TOPK = 50
NS = K_WAY + N_AUG

QT = 1024
NT1 = NQ // QT
TAIL = 32

_NOISE = jax.random.normal(jax.random.key(42), (K_WAY, N_AUG, D),
                           dtype=jnp.float32)


def _logsumexp_rows(z):
    m = jnp.max(z, axis=1, keepdims=True)
    return m + jnp.log(jnp.sum(jnp.exp(z - m), axis=1, keepdims=True))


def _accumulate(q, tgt, proto, pn, valid, stats_scr, cnt_scr, loss_ref):
    rows = q.shape[0]
    qsq = q * q
    qn = jnp.sum(qsq, axis=1, keepdims=True)
    qp = lax.dot_general(q, proto, (((1,), (1,)), ((), ())),
                         preferred_element_type=jnp.float32)
    d2 = qn + pn[None, :] - 2.0 * qp
    d_e = jnp.sqrt(jnp.clip(d2, 1e-12))

    z = -d_e
    zmax = jnp.max(z, axis=1, keepdims=True)
    iota_k = lax.broadcasted_iota(jnp.int32, (rows, K_WAY), 1)
    cand = jnp.where(z >= zmax, iota_k, K_WAY)
    pred = jnp.min(cand, axis=1, keepdims=True)
    onehot = jnp.where((iota_k == pred) & valid, 1.0, 0.0)

    cnt_scr[...] += jnp.sum(onehot, axis=0, keepdims=True)
    stats_scr[0] += lax.dot_general(onehot, q, (((0,), (0,)), ((), ())),
                                    preferred_element_type=jnp.float32)
    stats_scr[1] += lax.dot_general(onehot, qsq, (((0,), (0,)), ((), ())),
                                    preferred_element_type=jnp.float32)

    a = z - _logsumexp_rows(z)
    b = a - _logsumexp_rows(a)
    tgt_f = tgt.astype(jnp.float32)
    loss_ref[...] += -jnp.sum(tgt_f * b, keepdims=True).reshape(1, 1)


def _refresh_protos(sup_ref, stats_scr, cnt_scr, noise_ref, pnew_scr):
    sup = sup_ref[:NSUP, :].reshape(K_WAY, N_SHOT, D)
    proto = jnp.mean(sup, axis=1)
    cnt = cnt_scr[...].reshape(K_WAY, 1) + float(N_SHOT)
    s_sum = jnp.sum(sup, axis=1) + stats_scr[0]
    s_sq = jnp.sum(sup * sup, axis=1) + stats_scr[1]
    mean_c = s_sum / cnt
    var_c = (s_sq - cnt * mean_c * mean_c) / (cnt - 1.0)
    std_c = jnp.sqrt(jnp.clip(var_c, 1e-12))

    noise = noise_ref[...]
    delta = mean_c - proto
    u = delta * std_c
    v = std_c * std_c
    dn2 = jnp.sum(delta * delta, axis=1, keepdims=True)
    bdims = (((2,), (1,)), ((0,), (0,)))
    cross = lax.dot_general(noise, u, bdims,
                            preferred_element_type=jnp.float32)
    quad = lax.dot_general(noise * noise, v, bdims,
                           preferred_element_type=jnp.float32)
    d2_smp = dn2 + 2.0 * cross + quad

    dsup = sup - proto[:, None, :]
    d2_sup = jnp.sum(dsup * dsup, axis=2)
    d = jnp.sqrt(jnp.clip(jnp.concatenate([d2_sup, d2_smp], axis=1),
                          1e-12))

    di = d[:, :, None]
    dj = d[:, None, :]
    ii = lax.broadcasted_iota(jnp.int32, (K_WAY, NS, NS), 1)
    jj = lax.broadcasted_iota(jnp.int32, (K_WAY, NS, NS), 2)
    cmp = (dj < di) | ((dj == di) & (jj < ii))
    rank = jnp.sum(cmp.astype(jnp.float32), axis=2)
    sel = (rank < float(TOPK)).astype(jnp.float32)
    sel_sup = sel[:, :N_SHOT]
    sel_smp = sel[:, N_SHOT:]
    nsel = jnp.sum(sel_smp, axis=1, keepdims=True)

    sup_part = lax.dot_general(sel_sup, sup, (((1,), (1,)), ((0,), (0,))),
                               preferred_element_type=jnp.float32)
    noise_part = lax.dot_general(sel_smp, noise, (((1,), (1,)), ((0,), (0,))),
                                 preferred_element_type=jnp.float32)
    pnew = sup_part + nsel * mean_c + std_c * noise_part
    pnew_scr[:K_WAY] = pnew * (1.0 / TOPK)


def _softmax_rows(q, pnew, pn):
    qn = jnp.sum(q * q, axis=1, keepdims=True)
    qp = lax.dot_general(q, pnew, (((1,), (1,)), ((), ())),
                         preferred_element_type=jnp.float32)
    d2 = qn + pn[None, :] - 2.0 * qp
    z = -jnp.sqrt(jnp.clip(d2, 1e-12))
    m = jnp.max(z, axis=1, keepdims=True)
    e = jnp.exp(z - m)
    return e / jnp.sum(e, axis=1, keepdims=True)


def _body(q_ref, tailf_ref, tgt_ref, tgtt_ref, noise_ref,
          loss_ref, y_ref, yt_ref,
          proto_scr, stats_scr, cnt_scr, pnew_scr):
    i = pl.program_id(0)

    @pl.when(i == 0)
    def _init():
        stats_scr[...] = jnp.zeros_like(stats_scr)
        cnt_scr[...] = jnp.zeros_like(cnt_scr)
        loss_ref[...] = jnp.zeros_like(loss_ref)
        sup = q_ref[:NSUP, :].reshape(K_WAY, N_SHOT, D)
        proto_scr[:K_WAY] = jnp.mean(sup, axis=1)

    @pl.when(i < NT1)
    def _phase1():
        proto = proto_scr[:K_WAY]
        pn = jnp.sum(proto * proto, axis=1)
        q = q_ref[...]
        r0 = lax.broadcasted_iota(jnp.int32, (QT, 1), 0) + i * QT
        valid = r0 >= NSUP
        _accumulate(q, tgt_ref[...], proto, pn, valid,
                    stats_scr, cnt_scr, loss_ref)

        @pl.when(i == NT1 - 1)
        def _tail():
            rloc = lax.broadcasted_iota(jnp.int32, (TAIL, 1), 0)
            tvalid = rloc < NSUP
            qt = jnp.where(tvalid, tailf_ref[...], 0.0)
            _accumulate(qt, tgtt_ref[...], proto, pn, tvalid,
                        stats_scr, cnt_scr, loss_ref)

    @pl.when(i == NT1)
    def _mid():
        _refresh_protos(q_ref, stats_scr, cnt_scr, noise_ref, pnew_scr)

    @pl.when(i >= NT1)
    def _phase2():
        pnew = pnew_scr[:K_WAY]
        pn = jnp.sum(pnew * pnew, axis=1)
        y_ref[...] = _softmax_rows(q_ref[...], pnew, pn)

        @pl.when(i == 2 * NT1 - 1)
        def _tail2():
            yt_ref[...] = _softmax_rows(tailf_ref[...], pnew, pn)


@jax.jit
def kernel(feat, label):
    tgt_oh = jax.nn.one_hot(label[1], K_WAY, dtype=jnp.int8)
    tgt_pad = jnp.pad(tgt_oh, ((NSUP, TAIL - NSUP), (0, 0)))

    loss_sum, y_main, y_tail = pl.pallas_call(
        _body,
        grid=(2 * NT1,),
        in_specs=[
            pl.BlockSpec((QT, D),
                         lambda i: (jnp.where(i < NT1, i, i - NT1), 0)),
            pl.BlockSpec((TAIL, D), lambda i: (NQ // TAIL, 0)),
            pl.BlockSpec((QT, K_WAY), lambda i: (jnp.minimum(i, NT1 - 1), 0)),
            pl.BlockSpec((TAIL, K_WAY), lambda i: (NQ // TAIL, 0)),
            pl.BlockSpec((K_WAY, N_AUG, D), lambda i: (0, 0, 0)),
        ],
        out_specs=[
            pl.BlockSpec((1, 1), lambda i: (0, 0)),
            pl.BlockSpec((QT, K_WAY),
                         lambda i: (jnp.where(i < NT1, 0, i - NT1), 0)),
            pl.BlockSpec((TAIL, K_WAY), lambda i: (0, 0)),
        ],
        out_shape=[
            jax.ShapeDtypeStruct((1, 1), jnp.float32),
            jax.ShapeDtypeStruct((NQ, K_WAY), jnp.float32),
            jax.ShapeDtypeStruct((TAIL, K_WAY), jnp.float32),
        ],
        scratch_shapes=[
            pltpu.VMEM((8, D), jnp.float32),
            pltpu.VMEM((2, K_WAY, D), jnp.float32),
            pltpu.VMEM((1, K_WAY), jnp.float32),
            pltpu.VMEM((8, D), jnp.float32),
        ],
    )(feat, feat, tgt_pad, tgt_pad, _NOISE)

    y_pred = jnp.concatenate(
        [lax.slice(y_main, (NSUP, 0), (NQ, K_WAY)),
         lax.slice(y_tail, (0, 0), (NSUP, K_WAY))], axis=0)
    loss = loss_sum[0, 0] / NQ
    return (y_pred, loss)
